# baseline (speedup 1.0000x reference)
import functools

import jax
import jax.numpy as jnp
from jax import lax
from jax.experimental import pallas as pl
from jax.experimental.pallas import tpu as pltpu
from jax.experimental.pallas import tpu_sc as plsc

D_MODEL = 64
SCALE = 8.0
NCORES = 2
NSUB = 16
NW = NCORES * NSUB
SUB = 128
CHUNK = 512
RSUB = CHUNK // SUB
LANES = 16


def _emb_body(x_hbm, table_hbm, out_hbm, idx_v, rows_v, gsem):
    nrows_w = x_hbm.shape[0] // NW
    rows_per_w = nrows_w * SUB
    nchunk = rows_per_w // CHUNK
    wid = lax.axis_index("s") * NCORES + lax.axis_index("c")
    base = wid * rows_per_w

    pltpu.sync_copy(x_hbm.at[pl.ds(wid * nrows_w, nrows_w)], idx_v)

    def chunk_body(g, carry):
        copies = []
        for j in range(RSUB):
            copies.append(pltpu.async_copy(
                table_hbm.at[idx_v.at[g * RSUB + j]],
                rows_v.at[pl.ds(j * SUB, SUB)],
                gsem,
            ))
        for c in copies:
            c.wait()

        def row_body(r, c2):
            for col in range(D_MODEL // LANES):
                sl = pl.ds(col * LANES, LANES)
                rows_v[r, sl] = rows_v[r, sl] * SCALE
            return c2
        lax.fori_loop(0, CHUNK, row_body, 0)

        pltpu.sync_copy(rows_v, out_hbm.at[pl.ds(base + g * CHUNK, CHUNK)])
        return carry

    lax.fori_loop(0, nchunk, chunk_body, 0)


@functools.partial(jax.jit, static_argnames=())
def kernel(x, table):
    b_total = x.size
    xf = x.reshape(b_total // SUB, SUB)
    mesh = plsc.VectorSubcoreMesh(core_axis_name="c", subcore_axis_name="s")
    nrows_w = xf.shape[0] // NW
    run = pl.kernel(
        _emb_body,
        mesh=mesh,
        out_type=jax.ShapeDtypeStruct((b_total, D_MODEL), jnp.float32),
        scratch_types=[
            pltpu.VMEM((nrows_w, SUB), jnp.int32),
            pltpu.VMEM((CHUNK, D_MODEL), jnp.float32),
            pltpu.SemaphoreType.DMA,
        ],
        compiler_params=pltpu.CompilerParams(use_tc_tiling_on_sc=False),
    )
    out = run(xf, table)
    return out.reshape(x.shape + (D_MODEL,))


# speedup vs baseline: 1.1130x; 1.1130x over previous
import functools

import jax
import jax.numpy as jnp
from jax import lax
from jax.experimental import pallas as pl
from jax.experimental.pallas import tpu as pltpu
from jax.experimental.pallas import tpu_sc as plsc

D_MODEL = 64
SCALE = 8.0
NCORES = 2
NSUB = 16
NW = NCORES * NSUB
SUB = 128
CHUNK = 256
RSUB = CHUNK // SUB
LANES = 16
RU = 8


def _fire_gathers(table_hbm, idx_v, gbuf, gsem, g):
    for j in range(RSUB):
        pltpu.async_copy(
            table_hbm.at[idx_v.at[g * RSUB + j]],
            gbuf.at[pl.ds(j * SUB, SUB)],
            gsem,
        )


def _wait_gathers(out_hbm, gbuf, gsem):
    pltpu.make_async_copy(out_hbm.at[pl.ds(0, CHUNK)], gbuf, gsem).wait()


def _scale_chunk(gbuf, obuf):
    def row_body(i, c):
        r0 = i * RU
        for u in range(RU):
            for col in range(D_MODEL // LANES):
                sl = pl.ds(col * LANES, LANES)
                obuf[r0 + u, sl] = gbuf[r0 + u, sl] * SCALE
        return c
    lax.fori_loop(0, CHUNK // RU, row_body, 0)


def _emb_body(x_hbm, table_hbm, out_hbm,
              idx_v, gbuf0, gbuf1, obuf0, obuf1,
              gsem0, gsem1, osem0, osem1):
    nrows_w = x_hbm.shape[0] // NW
    rows_per_w = nrows_w * SUB
    nchunk = rows_per_w // CHUNK
    wid = lax.axis_index("s") * NCORES + lax.axis_index("c")
    base = wid * rows_per_w

    gbufs = (gbuf0, gbuf1)
    obufs = (obuf0, obuf1)
    gsems = (gsem0, gsem1)
    osems = (osem0, osem1)

    pltpu.sync_copy(x_hbm.at[pl.ds(wid * nrows_w, nrows_w)], idx_v)

    for b in range(2):
        _fire_gathers(table_hbm, idx_v, gbufs[b], gsems[b], b)

    def body(i, carry):
        for b in range(2):
            g = 2 * i + b
            _wait_gathers(out_hbm, gbufs[b], gsems[b])

            @pl.when(i >= 1)
            def _():
                pltpu.make_async_copy(
                    obufs[b], out_hbm.at[pl.ds(base, CHUNK)], osems[b]).wait()

            _scale_chunk(gbufs[b], obufs[b])

            @pl.when(g + 2 < nchunk)
            def _():
                _fire_gathers(table_hbm, idx_v, gbufs[b], gsems[b], g + 2)

            pltpu.async_copy(
                obufs[b], out_hbm.at[pl.ds(base + g * CHUNK, CHUNK)], osems[b])
        return carry

    lax.fori_loop(0, nchunk // 2, body, 0)

    for b in range(2):
        pltpu.make_async_copy(
            obufs[b], out_hbm.at[pl.ds(base, CHUNK)], osems[b]).wait()


@functools.partial(jax.jit, static_argnames=())
def kernel(x, table):
    b_total = x.size
    xf = x.reshape(b_total // SUB, SUB)
    mesh = plsc.VectorSubcoreMesh(core_axis_name="c", subcore_axis_name="s")
    nrows_w = xf.shape[0] // NW
    run = pl.kernel(
        _emb_body,
        mesh=mesh,
        out_type=jax.ShapeDtypeStruct((b_total, D_MODEL), jnp.float32),
        scratch_types=[
            pltpu.VMEM((nrows_w, SUB), jnp.int32),
            pltpu.VMEM((CHUNK, D_MODEL), jnp.float32),
            pltpu.VMEM((CHUNK, D_MODEL), jnp.float32),
            pltpu.VMEM((CHUNK, D_MODEL), jnp.float32),
            pltpu.VMEM((CHUNK, D_MODEL), jnp.float32),
            pltpu.SemaphoreType.DMA,
            pltpu.SemaphoreType.DMA,
            pltpu.SemaphoreType.DMA,
            pltpu.SemaphoreType.DMA,
        ],
        compiler_params=pltpu.CompilerParams(use_tc_tiling_on_sc=False),
    )
    out = run(xf, table)
    return out.reshape(x.shape + (D_MODEL,))


# speedup vs baseline: 1.4864x; 1.3355x over previous
import functools

import jax
import jax.numpy as jnp
from jax import lax
from jax.experimental import pallas as pl
from jax.experimental.pallas import tpu as pltpu
from jax.experimental.pallas import tpu_sc as plsc

D_MODEL = 64
SCALE = 8.0
NCORES = 2
NSUB = 16
NW = NCORES * NSUB
SUB = 128
CHUNK = 256
RSUB = CHUNK // SUB
LANES = 16
RU = 8


def _fire_gathers(table_hbm, idx_v, gbuf, gsem, g):
    for j in range(RSUB):
        pltpu.async_copy(
            table_hbm.at[idx_v.at[g * RSUB + j]],
            gbuf.at[pl.ds(j * SUB, SUB)],
            gsem,
        )


def _wait_gathers(out_hbm, gbuf, gsem):
    pltpu.make_async_copy(
        out_hbm.at[pl.ds(0, CHUNK), pl.ds(0, D_MODEL)], gbuf, gsem).wait()


def _scale_chunk(gbuf, obuf):
    def row_body(i, c):
        r0 = i * RU
        for u in range(RU):
            for col in range(D_MODEL // LANES):
                sl = pl.ds(col * LANES, LANES)
                obuf[r0 + u, sl] = gbuf[r0 + u, sl] * SCALE
        return c
    lax.fori_loop(0, CHUNK // RU, row_body, 0)


def _emb_body(x_hbm, table_hbm, out_hbm,
              idx_v, gbuf0, gbuf1, obuf0, obuf1,
              gsem0, gsem1, osem0, osem1):
    nrows_w = x_hbm.shape[0] // NW
    rows_per_w = nrows_w * SUB
    nchunk = rows_per_w // CHUNK
    wid = lax.axis_index("s") * NCORES + lax.axis_index("c")
    base = wid * rows_per_w

    gbufs = (gbuf0, gbuf1)
    obufs = (obuf0, obuf1)
    gsems = (gsem0, gsem1)
    osems = (osem0, osem1)

    pltpu.sync_copy(x_hbm.at[pl.ds(wid * nrows_w, nrows_w)], idx_v)

    for b in range(2):
        _fire_gathers(table_hbm, idx_v, gbufs[b], gsems[b], b)

    def body(i, carry):
        for b in range(2):
            g = 2 * i + b
            _wait_gathers(out_hbm, gbufs[b], gsems[b])

            @pl.when(i >= 1)
            def _():
                pltpu.make_async_copy(
                    obufs[b],
                    out_hbm.at[pl.ds(base, CHUNK), pl.ds(0, D_MODEL)],
                    osems[b]).wait()

            _scale_chunk(gbufs[b], obufs[b])

            @pl.when(g + 2 < nchunk)
            def _():
                _fire_gathers(table_hbm, idx_v, gbufs[b], gsems[b], g + 2)

            pltpu.async_copy(
                obufs[b],
                out_hbm.at[pl.ds(base + g * CHUNK, CHUNK), pl.ds(0, D_MODEL)],
                osems[b])
        return carry

    lax.fori_loop(0, nchunk // 2, body, 0)

    for b in range(2):
        pltpu.make_async_copy(
            obufs[b],
            out_hbm.at[pl.ds(base, CHUNK), pl.ds(0, D_MODEL)],
            osems[b]).wait()


@functools.partial(jax.jit, static_argnames=())
def kernel(x, table):
    b_total = x.size
    xf = x.reshape(b_total // SUB, SUB)
    mesh = plsc.VectorSubcoreMesh(core_axis_name="c", subcore_axis_name="s")
    nrows_w = xf.shape[0] // NW
    run = pl.kernel(
        _emb_body,
        mesh=mesh,
        out_type=jax.ShapeDtypeStruct((b_total, 2 * D_MODEL), jnp.float32),
        scratch_types=[
            pltpu.VMEM((nrows_w, SUB), jnp.int32),
            pltpu.VMEM((CHUNK, D_MODEL), jnp.float32),
            pltpu.VMEM((CHUNK, D_MODEL), jnp.float32),
            pltpu.VMEM((CHUNK, D_MODEL), jnp.float32),
            pltpu.VMEM((CHUNK, D_MODEL), jnp.float32),
            pltpu.SemaphoreType.DMA,
            pltpu.SemaphoreType.DMA,
            pltpu.SemaphoreType.DMA,
            pltpu.SemaphoreType.DMA,
        ],
        compiler_params=pltpu.CompilerParams(use_tc_tiling_on_sc=False),
    )
    out = run(xf, table)
    return out[:, :D_MODEL].reshape(x.shape + (D_MODEL,))
